# Initial kernel scaffold; baseline (speedup 1.0000x reference)
import jax
import jax.numpy as jnp
from jax.experimental import pallas as pl


def kernel(x, edge_index, edge_attr, batch, Wt1, bt1, Wg1, bg1, Wt2, bt2, Wg2, bg2, Wc, bc, Wf, bf):
    raise NotImplementedError("write your pallas kernel here")


# speedup vs baseline: 7.0808x; 7.0808x over previous
import functools

import jax
import jax.numpy as jnp
from jax import lax
from jax.experimental import pallas as pl
from jax.experimental.pallas import tpu as pltpu
from jax.experimental.pallas import tpu_sc as plsc

_N = 2048
_W = 35
_C1 = 128
_C2 = 64
_E = 16384
_BS = 256
_EC = 8

_LANES = 16
_ROWS_PER_PASS = 32


def _adj_body(dst_hbm, src_hbm, ea_hbm, a_hbm, dstv, srcv, attrv, chunk):
    cid = lax.axis_index("c")
    sid = lax.axis_index("s")
    wid = sid * 2 + cid
    pltpu.sync_copy(dst_hbm, dstv)
    pltpu.sync_copy(src_hbm, srcv)
    pltpu.sync_copy(ea_hbm, attrv)
    zeros16 = jnp.zeros((_LANES,), jnp.float32)

    for p in range(2):
        base = wid * 64 + p * _ROWS_PER_PASS

        def zero_body(i, _, chunk=chunk):
            chunk[pl.ds(i * _LANES, _LANES)] = zeros16
            return 0

        lax.fori_loop(0, _ROWS_PER_PASS * (_N // _LANES), zero_body, 0)

        def edge_body(i, _, base=base, chunk=chunk):
            d = dstv[pl.ds(i * _LANES, _LANES)]
            inr = (d >= base) & (d < base + _ROWS_PER_PASS)
            s = srcv[pl.ds(i * _LANES, _LANES)]
            a = attrv[pl.ds(i * _LANES, _LANES)]
            flat = jnp.where(inr, (d - base) * _N + s, 0)
            plsc.addupdate_scatter(chunk, [flat], a, mask=inr)
            return 0

        lax.fori_loop(0, _E // _LANES, edge_body, 0)
        pltpu.sync_copy(
            chunk, a_hbm.at[pl.ds(base * _N, _ROWS_PER_PASS * _N)])


def _build_adj(edge_index, edge_attr):
    mesh = plsc.VectorSubcoreMesh(core_axis_name="c", subcore_axis_name="s")
    kern = functools.partial(
        pl.kernel,
        out_type=jax.ShapeDtypeStruct((_N * _N,), jnp.float32),
        mesh=mesh,
        scratch_types=[
            pltpu.VMEM((_E,), jnp.int32),
            pltpu.VMEM((_E,), jnp.int32),
            pltpu.VMEM((_E,), jnp.float32),
            pltpu.VMEM((_ROWS_PER_PASS * _N,), jnp.float32),
        ],
        compiler_params=pltpu.CompilerParams(
            needs_layout_passes=False, use_tc_tiling_on_sc=False),
    )(_adj_body)
    return kern(edge_index[1], edge_index[0], edge_attr).reshape(_N, _N)


def _tconv_body(x_ref, w_ref, b_ref, o_ref):
    xb = x_ref[...]
    bn, w, cin = xb.shape
    cout = o_ref.shape[2]
    zero = jnp.zeros((bn, 1, cin), jnp.float32)
    xp = jnp.concatenate([zero, xb, zero], axis=1)
    acc = jnp.broadcast_to(b_ref[...], (bn * w, cout))
    for k in range(3):
        acc = acc + jnp.dot(
            xp[:, k:k + w, :].reshape(bn * w, cin), w_ref[k],
            preferred_element_type=jnp.float32)
    o_ref[...] = jnp.maximum(acc, 0.0).reshape(bn, w, cout)


def _tconv(x3, wk, b):
    n, w, cin = x3.shape
    cout = wk.shape[2]
    bn = 256
    return pl.pallas_call(
        _tconv_body,
        grid=(n // bn,),
        in_specs=[
            pl.BlockSpec((bn, w, cin), lambda i: (i, 0, 0)),
            pl.BlockSpec((3, cin, cout), lambda i: (0, 0, 0)),
            pl.BlockSpec((1, cout), lambda i: (0, 0)),
        ],
        out_specs=pl.BlockSpec((bn, w, cout), lambda i: (i, 0, 0)),
        out_shape=jax.ShapeDtypeStruct((n, w, cout), jnp.float32),
    )(x3, wk, b.reshape(1, cout))


def _aggmm_body(a_ref, x_ref, o_ref):
    k = pl.program_id(1)

    @pl.when(k == 0)
    def _():
        o_ref[...] = jnp.zeros_like(o_ref)

    o_ref[...] += jnp.dot(a_ref[...], x_ref[...],
                          preferred_element_type=jnp.float32)


def _aggmm(adj, tf):
    n, f = tf.shape
    bn = 512
    nb = n // bn
    return pl.pallas_call(
        _aggmm_body,
        grid=(nb, nb),
        in_specs=[
            pl.BlockSpec((bn, bn), lambda i, k: (i, k)),
            pl.BlockSpec((bn, f), lambda i, k: (k, 0)),
        ],
        out_specs=pl.BlockSpec((bn, f), lambda i, k: (i, 0)),
        out_shape=jax.ShapeDtypeStruct((n, f), jnp.float32),
    )(adj, tf)


def _mix_body(t_ref, agg_ref, g_ref, b_ref, o_ref):
    bn, w, c = t_ref.shape
    h = (t_ref[...] + agg_ref[...]).reshape(bn * w, c)
    hm = jnp.maximum(
        jnp.dot(h, g_ref[...], preferred_element_type=jnp.float32)
        + b_ref[...], 0.0)
    o_ref[...] = hm.reshape(bn, w, c)


def _mix(t, agg3, g, b):
    n, w, c = t.shape
    bn = 256
    return pl.pallas_call(
        _mix_body,
        grid=(n // bn,),
        in_specs=[
            pl.BlockSpec((bn, w, c), lambda i: (i, 0, 0)),
            pl.BlockSpec((bn, w, c), lambda i: (i, 0, 0)),
            pl.BlockSpec((c, c), lambda i: (0, 0)),
            pl.BlockSpec((1, c), lambda i: (0, 0)),
        ],
        out_specs=pl.BlockSpec((bn, w, c), lambda i: (i, 0, 0)),
        out_shape=jax.ShapeDtypeStruct((n, w, c), jnp.float32),
    )(t, agg3, g, b.reshape(1, c))


def _matmul_bias_body(x_ref, w_ref, b_ref, o_ref, *, relu):
    r = jnp.dot(x_ref[...], w_ref[...],
                preferred_element_type=jnp.float32) + b_ref[...]
    if relu:
        r = jnp.maximum(r, 0.0)
    o_ref[...] = r


def _matmul_bias(x, w, b, bn, relu):
    n, f = x.shape
    c = w.shape[1]
    return pl.pallas_call(
        functools.partial(_matmul_bias_body, relu=relu),
        grid=(n // bn,),
        in_specs=[
            pl.BlockSpec((bn, f), lambda i: (i, 0)),
            pl.BlockSpec((f, c), lambda i: (0, 0)),
            pl.BlockSpec((1, c), lambda i: (0, 0)),
        ],
        out_specs=pl.BlockSpec((bn, c), lambda i: (i, 0)),
        out_shape=jax.ShapeDtypeStruct((n, c), jnp.float32),
    )(x, w, b)


def kernel(x, edge_index, edge_attr, batch, Wt1, bt1, Wg1, bg1,
           Wt2, bt2, Wg2, bg2, Wc, bc, Wf, bf):
    del batch
    w1 = jnp.transpose(Wt1, (2, 1, 0))
    w2 = jnp.transpose(Wt2, (2, 1, 0))
    g1 = Wg1.T
    g2 = Wg2.T
    wck = jnp.transpose(Wc, (2, 1, 0))
    cols = []
    for t in range(4):
        col = jnp.zeros((_W, _C2, _C2), jnp.float32)
        col = lax.dynamic_update_slice(col, wck, (t, 0, 0))
        cols.append(col.reshape(_W * _C2, _C2))
    wcbig = jnp.concatenate(cols, axis=1)
    bctile = jnp.tile(bc, (4,)).reshape(1, 4 * _C2)
    wf2 = Wf.reshape(_EC, _C2, 4).transpose(0, 2, 1).reshape(_EC * 4 * _C2, 1)

    adj = _build_adj(edge_index, edge_attr)
    x3 = x.reshape(_N, _W, _C1)
    t1 = _tconv(x3, w1, bt1)
    agg1 = _aggmm(adj, t1.reshape(_N, _W * _C1))
    m1 = _mix(t1, agg1.reshape(_N, _W, _C1), g1, bg1)
    t2 = _tconv(m1, w2, bt2)
    agg2 = _aggmm(adj, t2.reshape(_N, _W * _C2))
    m2 = _mix(t2, agg2.reshape(_N, _W, _C2), g2, bg2)
    c3 = _matmul_bias(m2.reshape(_N, _W * _C2), wcbig, bctile,
                      bn=256, relu=False)
    out = _matmul_bias(c3.reshape(_BS, _EC * 4 * _C2), wf2,
                       bf.reshape(1, 1), bn=_BS, relu=True)
    return out
